# baseline (speedup 1.0000x reference)
import functools

import jax
import jax.numpy as jnp
from jax import lax
from jax.experimental import pallas as pl
from jax.experimental.pallas import tpu as pltpu
from jax.experimental.pallas import tpu_sc as plsc

_POOL = 4
_K = 16
_NW = 32


def _rup(n, m):
    return ((n + m - 1) // m) * m


def _knn_body(q4_ref, rct_ref, out_ref, *, k, nr):
    bq = out_ref.shape[0]
    q4 = q4_ref[...]
    d = None
    for dim in range(3):
        diff = q4[:, dim][:, None] - rct_ref[dim, :][None, :]
        sq = diff * diff
        d = sq if d is None else d + sq
    iota = lax.broadcasted_iota(jnp.int32, (bq, nr), 1)
    cols = []
    for j in range(k):
        m = jnp.min(d, axis=1, keepdims=True)
        eq = d == m
        cols.append(jnp.min(jnp.where(eq, iota, jnp.int32(nr)), axis=1)[:, None])
        if j + 1 < k:
            d = jnp.where(eq, jnp.float32(jnp.inf), d)
    out_ref[...] = jnp.concatenate(cols, axis=1) if k > 1 else cols[0]


def _knn(q4, rt, k, bq=256):
    nq, nr = q4.shape[0], rt.shape[1]
    return pl.pallas_call(
        functools.partial(_knn_body, k=k, nr=nr),
        grid=(nq // bq,),
        in_specs=[pl.BlockSpec((bq, 4), lambda i: (i, 0)),
                  pl.BlockSpec((4, nr), lambda i: (0, 0))],
        out_specs=pl.BlockSpec((bq, k), lambda i: (i, 0)),
        out_shape=jax.ShapeDtypeStruct((nq, k), jnp.int32),
    )(q4, rt)


def _pad_coords_t(c, n_pad, fill):
    n = c.shape[0]
    ct = jnp.concatenate([c.T, jnp.zeros((1, n), jnp.float32)], axis=0)
    return jnp.pad(ct, ((0, 0), (0, n_pad - n)), constant_values=fill)


def _mm_body(x_ref, w_ref, b_ref, o_ref, *, relu):
    acc = jnp.dot(x_ref[...], w_ref[...],
                  preferred_element_type=jnp.float32) + b_ref[...]
    o_ref[...] = jnp.maximum(acc, 0.0) if relu else acc


def _mm(x, w, b, relu=False, bm=256):
    m, kd = x.shape
    n = w.shape[1]
    return pl.pallas_call(
        functools.partial(_mm_body, relu=relu),
        grid=(m // bm,),
        in_specs=[pl.BlockSpec((bm, kd), lambda i: (i, 0)),
                  pl.BlockSpec((kd, n), lambda i: (0, 0)),
                  pl.BlockSpec((1, n), lambda i: (0, 0))],
        out_specs=pl.BlockSpec((bm, n), lambda i: (i, 0)),
        out_shape=jax.ShapeDtypeStruct((m, n), jnp.float32),
    )(x, w, b.reshape(1, -1))


def _glayer_body(x_ref, c_ref, wf_ref, wr_ref, b_ref, g_ref, qp_ref):
    t = jnp.dot(c_ref[...], wr_ref[...], preferred_element_type=jnp.float32)
    g = jnp.dot(x_ref[...], wf_ref[...], preferred_element_type=jnp.float32)
    g_ref[...] = g + t + b_ref[...]
    qp_ref[...] = t


def _glayer(x, c4, w, b, bm=256):
    m, kd = x.shape
    n = w.shape[1]
    wf = w[:kd]
    wr = jnp.pad(w[kd:kd + 3], ((0, 1), (0, 0)))
    return pl.pallas_call(
        _glayer_body,
        grid=(m // bm,),
        in_specs=[pl.BlockSpec((bm, kd), lambda i: (i, 0)),
                  pl.BlockSpec((bm, 4), lambda i: (i, 0)),
                  pl.BlockSpec((kd, n), lambda i: (0, 0)),
                  pl.BlockSpec((4, n), lambda i: (0, 0)),
                  pl.BlockSpec((1, n), lambda i: (0, 0))],
        out_specs=[pl.BlockSpec((bm, n), lambda i: (i, 0)),
                   pl.BlockSpec((bm, n), lambda i: (i, 0))],
        out_shape=[jax.ShapeDtypeStruct((m, n), jnp.float32),
                   jax.ShapeDtypeStruct((m, n), jnp.float32)],
    )(x, c4, wf, wr, b.reshape(1, -1))


def _pad_coords4(c, n_pad):
    n = c.shape[0]
    return jnp.pad(c, ((0, n_pad - n), (0, 1)))


def _pick_gq(nq_pad, k):
    nqw = nq_pad // _NW
    for g in range(min(128 // k, nqw), 0, -1):
        if nqw % g == 0 and (nqw // g) % 2 == 0 and (g * k) % 8 == 0:
            return g
    raise ValueError((nq_pad, k))


def _sc_gather_max(g, idx_flat, qp, k, gq):
    nq_pad, c = qp.shape
    nqw = nq_pad // _NW
    ngroups = nqw // gq
    nchunk = c // 16
    mesh = plsc.VectorSubcoreMesh(core_axis_name="c", subcore_axis_name="s")

    @functools.partial(
        pl.kernel, mesh=mesh,
        out_type=jax.ShapeDtypeStruct((nq_pad, c), jnp.float32),
        scratch_types=[
            pltpu.VMEM((nqw // gq, gq * k), jnp.int32),
            pltpu.VMEM((gq * k, c), jnp.float32),
            pltpu.VMEM((gq * k, c), jnp.float32),
            pltpu.VMEM((gq, c), jnp.float32),
            pltpu.VMEM((gq, c), jnp.float32),
            pltpu.VMEM((gq, c), jnp.float32),
            pltpu.SemaphoreType.DMA,
            pltpu.SemaphoreType.DMA,
            pltpu.SemaphoreType.DMA,
            pltpu.SemaphoreType.DMA,
        ],
    )
    def run(g_hbm, idx_hbm, qp_hbm, out_hbm, idx_v, rows0, rows1, qp0, qp1,
            out_v, gs0, gs1, qs0, qs1):
        wid = lax.axis_index("s") * 2 + lax.axis_index("c")
        base_q = wid * nqw
        rows = (rows0, rows1)
        qpb = (qp0, qp1)
        gsem = (gs0, gs1)
        qsem = (qs0, qs1)

        pltpu.sync_copy(idx_hbm.at[wid], idx_v)

        def start(gi, b):
            pltpu.async_copy(g_hbm.at[idx_v.at[gi]], rows[b], gsem[b])
            pltpu.async_copy(qp_hbm.at[pl.ds(base_q + gi * gq, gq)],
                             qpb[b], qsem[b])

        start(0, 0)
        start(1, 1)

        def pair(pi, _):
            for b in range(2):
                gi = pi * 2 + b
                pltpu.make_async_copy(
                    g_hbm.at[pl.ds(0, gq * k)], rows[b], gsem[b]).wait()
                pltpu.make_async_copy(
                    qp_hbm.at[pl.ds(0, gq)], qpb[b], qsem[b]).wait()

                def qloop(qq, _):
                    def cloop(cc, _):
                        sl = pl.ds(cc * 16, 16)
                        acc = rows[b][qq * k, sl]
                        for kk in range(1, k):
                            acc = jnp.maximum(acc, rows[b][qq * k + kk, sl])
                        out_v[qq, sl] = jnp.maximum(acc - qpb[b][qq, sl], 0.0)
                        return 0
                    return lax.fori_loop(0, nchunk, cloop, 0)

                lax.fori_loop(0, gq, qloop, 0)
                pltpu.sync_copy(out_v, out_hbm.at[pl.ds(base_q + gi * gq, gq)])

                @pl.when(gi + 2 < ngroups)
                def _():
                    start(gi + 2, b)
            return 0

        lax.fori_loop(0, ngroups // 2, pair, 0)

    return run(g, idx_flat.reshape(_NW, ngroups, gq * k), qp)


def kernel(x, params):
    n0 = x.shape[0]
    coords = [x[:, :3]]
    for _ in range(2):
        coords.append(coords[-1][::_POOL])
    n = [c.shape[0] for c in coords]
    npad = [_rup(v, 512) for v in n]
    feats = x[:, 3:]

    c4 = [_pad_coords4(c, p) for c, p in zip(coords, npad)]
    rts = [_pad_coords_t(c, _rup(v, 128), 1e18) for c, v in zip(coords, n)]
    neigh = [_knn(c4[i], rts[i], _K) for i in range(3)]
    up = [_knn(c4[i], rts[i + 1], 1) for i in range(2)]
    sub = [neigh[i][::_POOL] for i in range(2)]

    def pad_rows(a, rows):
        return jnp.pad(a, ((0, rows - a.shape[0]), (0, 0)))

    enc, enc_b = params["enc"], params["enc_b"]
    pool, pool_b = params["pool"], params["pool_b"]
    dec, dec_b = params["dec"], params["dec_b"]

    level_feats = []
    X = pad_rows(feats, npad[0])
    for i in range(3):
        for l in range(i + 1):
            G, Qp = _glayer(X, c4[i], enc[i][l], enc_b[i][l])
            X = _sc_gather_max(G, neigh[i].reshape(-1), Qp, _K,
                               _pick_gq(npad[i], _K))
        level_feats.append(X)
        if i < 2:
            G, Qp = _glayer(X, c4[i], pool[i][0], pool_b[i][0])
            sub_idx = pad_rows(sub[i], npad[i + 1]).reshape(-1)
            Qp_sub = pad_rows(Qp[::_POOL], npad[i + 1])
            X = _sc_gather_max(G, sub_idx, Qp_sub, _K,
                               _pick_gq(npad[i + 1], _K))
            for l in range(1, i + 1):
                X = _mm(X, pool[i][l], pool_b[i][l], relu=True)

    F = level_feats[2]
    for i in (1, 0):
        cu = dec[i].shape[0] - level_feats[i].shape[1]
        A = _mm(F, dec[i][:cu], jnp.zeros((dec[i].shape[1],), jnp.float32))
        Bneg = _mm(level_feats[i], -dec[i][cu:], -dec_b[i])
        F = _sc_gather_max(A, up[i].reshape(-1), Bneg, 1,
                           _pick_gq(npad[i], 1))

    return F[:n0]


# speedup vs baseline: 1.1068x; 1.1068x over previous
import functools

import jax
import jax.numpy as jnp
from jax import lax
from jax.experimental import pallas as pl
from jax.experimental.pallas import tpu as pltpu
from jax.experimental.pallas import tpu_sc as plsc

_POOL = 4
_K = 16
_NW = 32


def _rup(n, m):
    return ((n + m - 1) // m) * m


def _knn_body(q4_ref, rct_ref, out_ref, d_ref, *, k, nr):
    bq = out_ref.shape[0]
    q4 = q4_ref[...]
    d = None
    for dim in range(3):
        diff = q4[:, dim][:, None] - rct_ref[dim, :][None, :]
        sq = diff * diff
        d = sq if d is None else d + sq
    d_ref[...] = d
    iota = lax.broadcasted_iota(jnp.int32, (bq, nr), 1)
    cols = []
    for j in range(k):
        d = d_ref[...]
        m = jnp.min(d, axis=1, keepdims=True)
        eq = d == m
        cols.append(jnp.min(jnp.where(eq, iota, jnp.int32(nr)), axis=1)[:, None])
        if j + 1 < k:
            d_ref[...] = jnp.where(eq, jnp.float32(jnp.inf), d)
    out_ref[...] = jnp.concatenate(cols, axis=1) if k > 1 else cols[0]


def _knn(q4, rt, k, bq=64):
    nq, nr = q4.shape[0], rt.shape[1]
    return pl.pallas_call(
        functools.partial(_knn_body, k=k, nr=nr),
        grid=(nq // bq,),
        in_specs=[pl.BlockSpec((bq, 4), lambda i: (i, 0)),
                  pl.BlockSpec((4, nr), lambda i: (0, 0))],
        out_specs=pl.BlockSpec((bq, k), lambda i: (i, 0)),
        out_shape=jax.ShapeDtypeStruct((nq, k), jnp.int32),
        scratch_shapes=[pltpu.VMEM((bq, nr), jnp.float32)],
    )(q4, rt)


def _pad_coords_t(c, n_pad, fill):
    n = c.shape[0]
    ct = jnp.concatenate([c.T, jnp.zeros((1, n), jnp.float32)], axis=0)
    return jnp.pad(ct, ((0, 0), (0, n_pad - n)), constant_values=fill)


def _mm_body(x_ref, w_ref, b_ref, o_ref, *, relu):
    acc = jnp.dot(x_ref[...], w_ref[...],
                  preferred_element_type=jnp.float32) + b_ref[...]
    o_ref[...] = jnp.maximum(acc, 0.0) if relu else acc


def _mm(x, w, b, relu=False, bm=256):
    m, kd = x.shape
    n = w.shape[1]
    return pl.pallas_call(
        functools.partial(_mm_body, relu=relu),
        grid=(m // bm,),
        in_specs=[pl.BlockSpec((bm, kd), lambda i: (i, 0)),
                  pl.BlockSpec((kd, n), lambda i: (0, 0)),
                  pl.BlockSpec((1, n), lambda i: (0, 0))],
        out_specs=pl.BlockSpec((bm, n), lambda i: (i, 0)),
        out_shape=jax.ShapeDtypeStruct((m, n), jnp.float32),
    )(x, w, b.reshape(1, -1))


def _glayer_body(x_ref, c_ref, wf_ref, wr_ref, b_ref, g_ref, qp_ref):
    t = jnp.dot(c_ref[...], wr_ref[...], preferred_element_type=jnp.float32)
    g = jnp.dot(x_ref[...], wf_ref[...], preferred_element_type=jnp.float32)
    g_ref[...] = g + t + b_ref[...]
    qp_ref[...] = t


def _glayer(x, c4, w, b, bm=256):
    m, kd = x.shape
    n = w.shape[1]
    wf = w[:kd]
    wr = jnp.pad(w[kd:kd + 3], ((0, 1), (0, 0)))
    return pl.pallas_call(
        _glayer_body,
        grid=(m // bm,),
        in_specs=[pl.BlockSpec((bm, kd), lambda i: (i, 0)),
                  pl.BlockSpec((bm, 4), lambda i: (i, 0)),
                  pl.BlockSpec((kd, n), lambda i: (0, 0)),
                  pl.BlockSpec((4, n), lambda i: (0, 0)),
                  pl.BlockSpec((1, n), lambda i: (0, 0))],
        out_specs=[pl.BlockSpec((bm, n), lambda i: (i, 0)),
                   pl.BlockSpec((bm, n), lambda i: (i, 0))],
        out_shape=[jax.ShapeDtypeStruct((m, n), jnp.float32),
                   jax.ShapeDtypeStruct((m, n), jnp.float32)],
    )(x, c4, wf, wr, b.reshape(1, -1))


def _pad_coords4(c, n_pad):
    n = c.shape[0]
    return jnp.pad(c, ((0, n_pad - n), (0, 1)))


def _pick_gq(nq_pad, k):
    nqw = nq_pad // _NW
    for g in range(min(128 // k, nqw), 0, -1):
        if nqw % g == 0 and (nqw // g) % 2 == 0 and (g * k) % 8 == 0:
            return g
    raise ValueError((nq_pad, k))


def _sc_gather_max(g, idx_flat, qp, k, gq):
    nq_pad, c = qp.shape
    nqw = nq_pad // _NW
    ngroups = nqw // gq
    nchunk = c // 16
    mesh = plsc.VectorSubcoreMesh(core_axis_name="c", subcore_axis_name="s")

    @functools.partial(
        pl.kernel, mesh=mesh,
        out_type=jax.ShapeDtypeStruct((nq_pad, c), jnp.float32),
        scratch_types=[
            pltpu.VMEM((nqw // gq, gq * k), jnp.int32),
            pltpu.VMEM((gq * k, c), jnp.float32),
            pltpu.VMEM((gq * k, c), jnp.float32),
            pltpu.VMEM((gq, c), jnp.float32),
            pltpu.VMEM((gq, c), jnp.float32),
            pltpu.VMEM((gq, c), jnp.float32),
            pltpu.SemaphoreType.DMA,
            pltpu.SemaphoreType.DMA,
            pltpu.SemaphoreType.DMA,
            pltpu.SemaphoreType.DMA,
        ],
    )
    def run(g_hbm, idx_hbm, qp_hbm, out_hbm, idx_v, rows0, rows1, qp0, qp1,
            out_v, gs0, gs1, qs0, qs1):
        wid = lax.axis_index("s") * 2 + lax.axis_index("c")
        base_q = wid * nqw
        rows = (rows0, rows1)
        qpb = (qp0, qp1)
        gsem = (gs0, gs1)
        qsem = (qs0, qs1)

        pltpu.sync_copy(idx_hbm.at[wid], idx_v)

        def start(gi, b):
            pltpu.async_copy(g_hbm.at[idx_v.at[gi]], rows[b], gsem[b])
            pltpu.async_copy(qp_hbm.at[pl.ds(base_q + gi * gq, gq)],
                             qpb[b], qsem[b])

        start(0, 0)
        start(1, 1)

        def pair(pi, _):
            for b in range(2):
                gi = pi * 2 + b
                pltpu.make_async_copy(
                    g_hbm.at[pl.ds(0, gq * k)], rows[b], gsem[b]).wait()
                pltpu.make_async_copy(
                    qp_hbm.at[pl.ds(0, gq)], qpb[b], qsem[b]).wait()

                def qloop(qq, _):
                    base = qq * k
                    for cc in range(nchunk):
                        sl = pl.ds(cc * 16, 16)
                        acc = rows[b][base, sl]
                        for kk in range(1, k):
                            acc = jnp.maximum(acc, rows[b][base + kk, sl])
                        out_v[qq, sl] = jnp.maximum(acc - qpb[b][qq, sl], 0.0)
                    return 0

                lax.fori_loop(0, gq, qloop, 0)
                pltpu.sync_copy(out_v, out_hbm.at[pl.ds(base_q + gi * gq, gq)])

                @pl.when(gi + 2 < ngroups)
                def _():
                    start(gi + 2, b)
            return 0

        lax.fori_loop(0, ngroups // 2, pair, 0)

    return run(g, idx_flat.reshape(_NW, ngroups, gq * k), qp)


def kernel(x, params):
    n0 = x.shape[0]
    coords = [x[:, :3]]
    for _ in range(2):
        coords.append(coords[-1][::_POOL])
    n = [c.shape[0] for c in coords]
    npad = [_rup(v, 512) for v in n]
    feats = x[:, 3:]

    c4 = [_pad_coords4(c, p) for c, p in zip(coords, npad)]
    rts = [_pad_coords_t(c, _rup(v, 128), 1e18) for c, v in zip(coords, n)]
    neigh = [_knn(c4[i], rts[i], _K) for i in range(3)]
    up = [_knn(c4[i], rts[i + 1], 1) for i in range(2)]
    sub = [neigh[i][::_POOL] for i in range(2)]

    def pad_rows(a, rows):
        return jnp.pad(a, ((0, rows - a.shape[0]), (0, 0)))

    enc, enc_b = params["enc"], params["enc_b"]
    pool, pool_b = params["pool"], params["pool_b"]
    dec, dec_b = params["dec"], params["dec_b"]

    level_feats = []
    X = pad_rows(feats, npad[0])
    for i in range(3):
        for l in range(i + 1):
            G, Qp = _glayer(X, c4[i], enc[i][l], enc_b[i][l])
            X = _sc_gather_max(G, neigh[i].reshape(-1), Qp, _K,
                               _pick_gq(npad[i], _K))
        level_feats.append(X)
        if i < 2:
            G, Qp = _glayer(X, c4[i], pool[i][0], pool_b[i][0])
            sub_idx = pad_rows(sub[i], npad[i + 1]).reshape(-1)
            Qp_sub = pad_rows(Qp[::_POOL], npad[i + 1])
            X = _sc_gather_max(G, sub_idx, Qp_sub, _K,
                               _pick_gq(npad[i + 1], _K))
            for l in range(1, i + 1):
                X = _mm(X, pool[i][l], pool_b[i][l], relu=True)

    F = level_feats[2]
    for i in (1, 0):
        cu = dec[i].shape[0] - level_feats[i].shape[1]
        A = _mm(F, dec[i][:cu], jnp.zeros((dec[i].shape[1],), jnp.float32))
        Bneg = _mm(level_feats[i], -dec[i][cu:], -dec_b[i])
        F = _sc_gather_max(A, up[i].reshape(-1), Bneg, 1,
                           _pick_gq(npad[i], 1))

    return F[:n0]
